# baseline (speedup 1.0000x reference)
import functools

import jax
import jax.numpy as jnp
from jax import lax
from jax.experimental import pallas as pl
from jax.experimental.pallas import tpu as pltpu
from jax.experimental.pallas import tpu_sc as plsc

N = 10000
E = 160000
D_IN = 256
D_OUT = 512
DH = 128

NC = 2
NS = 16
CHUNK = 64
NBUF = 4
CPT = 160
G = CPT // NBUF
E_PAD = NS * CPT * CHUNK
EPT = CPT * CHUNK
AGG_ROWS = N + 240
ZSLAB = AGG_ROWS // NS
COPY_ROWS = 632


def _sc_aggregate(x2, srcl, dstl):
  mesh = plsc.VectorSubcoreMesh(core_axis_name="c", subcore_axis_name="s")

  @functools.partial(
      pl.kernel,
      out_type=(
          jax.ShapeDtypeStruct((2 * N, DH), jnp.float32),
          jax.ShapeDtypeStruct((N,), jnp.float32),
      ),
      mesh=mesh,
      scratch_types=[
          pltpu.VMEM_SHARED((AGG_ROWS, DH), jnp.float32),
          pltpu.VMEM_SHARED((AGG_ROWS,), jnp.float32),
          [pltpu.VMEM((NBUF * CHUNK,), jnp.int32)] * 2,
          [pltpu.VMEM((CHUNK,), jnp.int32)] * (2 * NBUF),
          [pltpu.VMEM((CHUNK, DH), jnp.float32)] * NBUF,
          pltpu.VMEM((CHUNK,), jnp.float32),
          pltpu.VMEM((ZSLAB,), jnp.float32),
          [pltpu.SemaphoreType.DMA] * NBUF,
          [pltpu.SemaphoreType.DMA] * NBUF,
          [pltpu.SemaphoreType.DMA] * 2,
          pltpu.SemaphoreType.DMA,
      ],
  )
  def k(x2_hbm, src_hbm, dst_hbm, agg2_hbm, deg_hbm,
        agg_sh, deg_sh, sbufs, dbufs, gbufs, ones_v, zrow,
        gsems, ssems, isems, dsem):
    c = lax.axis_index("c")
    s = lax.axis_index("s")
    gbuf = gbufs[0]

    sc_init = jax.named_scope("sc_init")
    sc_init.__enter__()
    def zero_gbuf_row(i, _):
      for kk in range(DH // 16):
        gbuf[i, pl.ds(kk * 16, 16)] = jnp.zeros((16,), jnp.float32)
      return _
    lax.fori_loop(0, CHUNK, zero_gbuf_row, None)
    for kk in range(ZSLAB // 16):
      zrow[pl.ds(kk * 16, 16)] = jnp.zeros((16,), jnp.float32)
    for kk in range(CHUNK // 16):
      ones_v[pl.ds(kk * 16, 16)] = jnp.ones((16,), jnp.float32)

    def zero_slab(j, _):
      pltpu.sync_copy(gbuf, agg_sh.at[pl.ds(s * ZSLAB + j * CHUNK, CHUNK)])
      return _
    lax.fori_loop(0, ZSLAB // CHUNK, zero_slab, None)
    pltpu.sync_copy(zrow, deg_sh.at[pl.ds(s * ZSLAB, ZSLAB)])

    plsc.subcore_barrier()
    sc_init.__exit__(None, None, None)

    def prefetch_idx(g, p):
      base = s * EPT + g * (NBUF * CHUNK)
      pltpu.async_copy(src_hbm.at[pl.ds(c * E_PAD + base, NBUF * CHUNK)],
                       sbufs[p], isems[p])
      for b in range(NBUF):
        pltpu.async_copy(dst_hbm.at[pl.ds(base + b * CHUNK, CHUNK)],
                         dbufs[p * NBUF + b], isems[p])

    def wait_idx(p):
      pltpu.make_async_copy(src_hbm.at[pl.ds(0, NBUF * CHUNK)],
                            sbufs[p], isems[p]).wait()
      for b in range(NBUF):
        pltpu.make_async_copy(dst_hbm.at[pl.ds(0, CHUNK)],
                              dbufs[p * NBUF + b], isems[p]).wait()

    def start_gather(p, b):
      pltpu.async_copy(x2_hbm.at[sbufs[p].at[pl.ds(b * CHUNK, CHUNK)]],
                       gbufs[b], gsems[b])

    def wait_gather(b):
      pltpu.make_async_copy(x2_hbm.at[sbufs[0].at[pl.ds(0, CHUNK)]],
                            gbufs[b], gsems[b]).wait()

    def start_scatter(p, b):
      pltpu.async_copy(gbufs[b], agg_sh.at[dbufs[p * NBUF + b]], ssems[b],
                       add=True)

    def wait_scatter(b):
      pltpu.make_async_copy(gbufs[b], agg_sh.at[dbufs[0]], ssems[b]).wait()

    sc_main = jax.named_scope("sc_main")
    sc_main.__enter__()
    prefetch_idx(0, 0)
    wait_idx(0)
    prefetch_idx(1, 1)
    for b in range(NBUF):
      start_gather(0, b)

    def pair(g2, _):
      for p in (0, 1):
        g = g2 * 2 + p
        for b in range(NBUF):
          wait_gather(b)
          start_scatter(p, b)

        @pl.when(c == 0)
        def _():
          for b in range(NBUF):
            pltpu.async_copy(ones_v, deg_sh.at[dbufs[p * NBUF + b]], dsem,
                             add=True)

        @pl.when(g + 1 < G)
        def _():
          wait_idx(1 - p)
          for b in range(NBUF):
            wait_scatter(b)
            start_gather(1 - p, b)

        @pl.when(c == 0)
        def _():
          for b in range(NBUF):
            pltpu.make_async_copy(ones_v, deg_sh.at[dbufs[0]], dsem).wait()

        @pl.when(g + 2 < G)
        def _():
          prefetch_idx(g + 2, p)
      return _
    lax.fori_loop(0, G // 2, pair, None)
    for b in range(NBUF):
      wait_scatter(b)

    plsc.subcore_barrier()
    sc_main.__exit__(None, None, None)

    @pl.when(s < NS - 1)
    def _():
      pltpu.sync_copy(agg_sh.at[pl.ds(s * COPY_ROWS, COPY_ROWS)],
                      agg2_hbm.at[pl.ds(c * N + s * COPY_ROWS, COPY_ROWS)])

      @pl.when(c == 0)
      def _():
        pltpu.sync_copy(deg_sh.at[pl.ds(s * COPY_ROWS, COPY_ROWS)],
                        zrow.at[pl.ds(0, COPY_ROWS)])
        pltpu.sync_copy(zrow.at[pl.ds(0, COPY_ROWS)],
                        deg_hbm.at[pl.ds(s * COPY_ROWS, COPY_ROWS)])

    @pl.when(s == NS - 1)
    def _():
      tail = N - (NS - 1) * COPY_ROWS
      pltpu.sync_copy(agg_sh.at[pl.ds((NS - 1) * COPY_ROWS, tail)],
                      agg2_hbm.at[pl.ds(c * N + (NS - 1) * COPY_ROWS, tail)])

      @pl.when(c == 0)
      def _():
        pltpu.sync_copy(deg_sh.at[pl.ds((NS - 1) * COPY_ROWS, tail)],
                        zrow.at[pl.ds(0, tail)])
        pltpu.sync_copy(zrow.at[pl.ds(0, tail)],
                        deg_hbm.at[pl.ds((NS - 1) * COPY_ROWS, tail)])

  return k(x2, srcl, dstl)


def _tc_body(x_ref, lo_ref, hi_ref, deg_ref, ws_ref, wn_ref, b_ref, out_ref):
  scale = 1.0 / jnp.maximum(deg_ref[...], 1.0)
  h = jnp.concatenate([lo_ref[...] * scale, hi_ref[...] * scale], axis=1)
  out = jnp.dot(x_ref[...], ws_ref[...], preferred_element_type=jnp.float32)
  out += jnp.dot(h, wn_ref[...], preferred_element_type=jnp.float32)
  out_ref[...] = out + b_ref[...]


def _tc_combine(in_feat, agg2, degc, W_self, W_neigh, b2):
  R = 1000
  grid = (N // R,)
  return pl.pallas_call(
      _tc_body,
      grid=grid,
      in_specs=[
          pl.BlockSpec((R, D_IN), lambda i: (i, 0)),
          pl.BlockSpec((R, DH), lambda i: (i, 0)),
          pl.BlockSpec((R, DH), lambda i: (i + N // R, 0)),
          pl.BlockSpec((R, 1), lambda i: (i, 0)),
          pl.BlockSpec((D_IN, D_OUT), lambda i: (0, 0)),
          pl.BlockSpec((D_IN, D_OUT), lambda i: (0, 0)),
          pl.BlockSpec((1, D_OUT), lambda i: (0, 0)),
      ],
      out_specs=pl.BlockSpec((R, D_OUT), lambda i: (i, 0)),
      out_shape=jax.ShapeDtypeStruct((N, D_OUT), jnp.float32),
  )(in_feat, agg2, agg2, degc, W_self, W_neigh, b2)


def kernel(in_feat, edge_index, W_self, W_neigh, b):
  src = edge_index[0]
  dst = edge_index[1]
  npad = E_PAD - E

  x2 = jnp.concatenate(
      [in_feat[:, :DH], in_feat[:, DH:], jnp.zeros((8, DH), jnp.float32)], axis=0)

  pad_src = jnp.full((npad,), 2 * N, jnp.int32)
  srcl = jnp.concatenate([src, pad_src, src + N, pad_src])
  dstl = jnp.concatenate([dst, jnp.full((npad,), N, jnp.int32)])

  agg2, deg = _sc_aggregate(x2, srcl, dstl)
  degc = deg.reshape(N, 1)
  b2 = b.reshape(1, D_OUT)
  return _tc_combine(in_feat, agg2, degc, W_self, W_neigh, b2)


# speedup vs baseline: 1.0010x; 1.0010x over previous
import functools

import jax
import jax.numpy as jnp
from jax import lax
from jax.experimental import pallas as pl
from jax.experimental.pallas import tpu as pltpu
from jax.experimental.pallas import tpu_sc as plsc

N = 10000
E = 160000
D_IN = 256
D_OUT = 512
DH = 128

NC = 2
NS = 16
CHUNK = 64
NBUF = 4
CPT = 160
G = CPT // NBUF
E_PAD = NS * CPT * CHUNK
EPT = CPT * CHUNK
AGG_ROWS = N + 240
ZSLAB = AGG_ROWS // NS
COPY_ROWS = 632


def _sc_aggregate(x2, srcl, dstl):
  mesh = plsc.VectorSubcoreMesh(core_axis_name="c", subcore_axis_name="s")

  @functools.partial(
      pl.kernel,
      out_type=(
          jax.ShapeDtypeStruct((2 * N, DH), jnp.float32),
          jax.ShapeDtypeStruct((N,), jnp.float32),
      ),
      mesh=mesh,
      scratch_types=[
          pltpu.VMEM_SHARED((AGG_ROWS, DH), jnp.float32),
          pltpu.VMEM_SHARED((AGG_ROWS,), jnp.float32),
          [pltpu.VMEM((NBUF * CHUNK,), jnp.int32)] * 2,
          [pltpu.VMEM((CHUNK,), jnp.int32)] * (2 * NBUF),
          [pltpu.VMEM((CHUNK, DH), jnp.float32)] * NBUF,
          pltpu.VMEM((CHUNK,), jnp.float32),
          pltpu.VMEM((ZSLAB,), jnp.float32),
          [pltpu.SemaphoreType.DMA] * NBUF,
          [pltpu.SemaphoreType.DMA] * NBUF,
          [pltpu.SemaphoreType.DMA] * 2,
          pltpu.SemaphoreType.DMA,
      ],
  )
  def k(x2_hbm, src_hbm, dst_hbm, agg2_hbm, deg_hbm,
        agg_sh, deg_sh, sbufs, dbufs, gbufs, ones_v, zrow,
        gsems, ssems, isems, dsem):
    c = lax.axis_index("c")
    s = lax.axis_index("s")
    gbuf = gbufs[0]

    def zero_gbuf_row(i, _):
      for kk in range(DH // 16):
        gbuf[i, pl.ds(kk * 16, 16)] = jnp.zeros((16,), jnp.float32)
      return _
    lax.fori_loop(0, CHUNK, zero_gbuf_row, None)
    for kk in range(ZSLAB // 16):
      zrow[pl.ds(kk * 16, 16)] = jnp.zeros((16,), jnp.float32)
    for kk in range(CHUNK // 16):
      ones_v[pl.ds(kk * 16, 16)] = jnp.ones((16,), jnp.float32)

    def zero_slab(j, _):
      pltpu.sync_copy(gbuf, agg_sh.at[pl.ds(s * ZSLAB + j * CHUNK, CHUNK)])
      return _
    lax.fori_loop(0, ZSLAB // CHUNK, zero_slab, None)
    pltpu.sync_copy(zrow, deg_sh.at[pl.ds(s * ZSLAB, ZSLAB)])

    plsc.subcore_barrier()

    def prefetch_idx(g, p):
      base = s * EPT + g * (NBUF * CHUNK)
      pltpu.async_copy(src_hbm.at[pl.ds(c * E_PAD + base, NBUF * CHUNK)],
                       sbufs[p], isems[p])
      for b in range(NBUF):
        pltpu.async_copy(dst_hbm.at[pl.ds(base + b * CHUNK, CHUNK)],
                         dbufs[p * NBUF + b], isems[p])

    def wait_idx(p):
      pltpu.make_async_copy(src_hbm.at[pl.ds(0, NBUF * CHUNK)],
                            sbufs[p], isems[p]).wait()
      for b in range(NBUF):
        pltpu.make_async_copy(dst_hbm.at[pl.ds(0, CHUNK)],
                              dbufs[p * NBUF + b], isems[p]).wait()

    def start_gather(p, b):
      pltpu.async_copy(x2_hbm.at[sbufs[p].at[pl.ds(b * CHUNK, CHUNK)]],
                       gbufs[b], gsems[b])

    def wait_gather(b):
      pltpu.make_async_copy(x2_hbm.at[sbufs[0].at[pl.ds(0, CHUNK)]],
                            gbufs[b], gsems[b]).wait()

    def start_scatter(p, b):
      pltpu.async_copy(gbufs[b], agg_sh.at[dbufs[p * NBUF + b]], ssems[b],
                       add=True)

    def wait_scatter(b):
      pltpu.make_async_copy(gbufs[b], agg_sh.at[dbufs[0]], ssems[b]).wait()

    prefetch_idx(0, 0)
    wait_idx(0)
    prefetch_idx(1, 1)
    for b in range(NBUF):
      start_gather(0, b)

    def pair(g2, _):
      for p in (0, 1):
        g = g2 * 2 + p
        for b in range(NBUF):
          wait_gather(b)
          start_scatter(p, b)

        @pl.when(c == 0)
        def _():
          for b in range(NBUF):
            pltpu.async_copy(ones_v, deg_sh.at[dbufs[p * NBUF + b]], dsem,
                             add=True)

        @pl.when(g + 1 < G)
        def _():
          wait_idx(1 - p)
          for b in range(NBUF):
            wait_scatter(b)
            start_gather(1 - p, b)

        @pl.when(c == 0)
        def _():
          for b in range(NBUF):
            pltpu.make_async_copy(ones_v, deg_sh.at[dbufs[0]], dsem).wait()

        @pl.when(g + 2 < G)
        def _():
          prefetch_idx(g + 2, p)
      return _
    lax.fori_loop(0, G // 2, pair, None)
    for b in range(NBUF):
      wait_scatter(b)

    plsc.subcore_barrier()

    @pl.when(s < NS - 1)
    def _():
      pltpu.sync_copy(agg_sh.at[pl.ds(s * COPY_ROWS, COPY_ROWS)],
                      agg2_hbm.at[pl.ds(c * N + s * COPY_ROWS, COPY_ROWS)])

      @pl.when(c == 0)
      def _():
        pltpu.sync_copy(deg_sh.at[pl.ds(s * COPY_ROWS, COPY_ROWS)],
                        zrow.at[pl.ds(0, COPY_ROWS)])
        pltpu.sync_copy(zrow.at[pl.ds(0, COPY_ROWS)],
                        deg_hbm.at[pl.ds(s * COPY_ROWS, COPY_ROWS)])

    @pl.when(s == NS - 1)
    def _():
      tail = N - (NS - 1) * COPY_ROWS
      pltpu.sync_copy(agg_sh.at[pl.ds((NS - 1) * COPY_ROWS, tail)],
                      agg2_hbm.at[pl.ds(c * N + (NS - 1) * COPY_ROWS, tail)])

      @pl.when(c == 0)
      def _():
        pltpu.sync_copy(deg_sh.at[pl.ds((NS - 1) * COPY_ROWS, tail)],
                        zrow.at[pl.ds(0, tail)])
        pltpu.sync_copy(zrow.at[pl.ds(0, tail)],
                        deg_hbm.at[pl.ds((NS - 1) * COPY_ROWS, tail)])

  return k(x2, srcl, dstl)


def _tc_body(x_ref, lo_ref, hi_ref, deg_ref, ws_ref, wn_ref, b_ref, out_ref):
  scale = 1.0 / jnp.maximum(deg_ref[...], 1.0)
  h = jnp.concatenate([lo_ref[...] * scale, hi_ref[...] * scale], axis=1)
  out = jnp.dot(x_ref[...], ws_ref[...], preferred_element_type=jnp.float32)
  out += jnp.dot(h, wn_ref[...], preferred_element_type=jnp.float32)
  out_ref[...] = out + b_ref[...]


def _tc_combine(in_feat, agg2, degc, W_self, W_neigh, b2):
  R = 1000
  grid = (N // R,)
  return pl.pallas_call(
      _tc_body,
      grid=grid,
      in_specs=[
          pl.BlockSpec((R, D_IN), lambda i: (i, 0)),
          pl.BlockSpec((R, DH), lambda i: (i, 0)),
          pl.BlockSpec((R, DH), lambda i: (i + N // R, 0)),
          pl.BlockSpec((R, 1), lambda i: (i, 0)),
          pl.BlockSpec((D_IN, D_OUT), lambda i: (0, 0)),
          pl.BlockSpec((D_IN, D_OUT), lambda i: (0, 0)),
          pl.BlockSpec((1, D_OUT), lambda i: (0, 0)),
      ],
      out_specs=pl.BlockSpec((R, D_OUT), lambda i: (i, 0)),
      out_shape=jax.ShapeDtypeStruct((N, D_OUT), jnp.float32),
  )(in_feat, agg2, agg2, degc, W_self, W_neigh, b2)


def kernel(in_feat, edge_index, W_self, W_neigh, b):
  src = edge_index[0]
  dst = edge_index[1]
  npad = E_PAD - E

  x2 = jnp.concatenate(
      [in_feat[:, :DH], in_feat[:, DH:], jnp.zeros((8, DH), jnp.float32)], axis=0)

  pad_src = jnp.full((npad,), 2 * N, jnp.int32)
  srcl = jnp.concatenate([src, pad_src, src + N, pad_src])
  dstl = jnp.concatenate([dst, jnp.full((npad,), N, jnp.int32)])

  agg2, deg = _sc_aggregate(x2, srcl, dstl)
  degc = deg.reshape(N, 1)
  b2 = b.reshape(1, D_OUT)
  return _tc_combine(in_feat, agg2, degc, W_self, W_neigh, b2)
